# Initial kernel scaffold; baseline (speedup 1.0000x reference)
import jax
import jax.numpy as jnp
from jax.experimental import pallas as pl


def kernel(x_atom_type, edge_index, batch, emb, W1a, b1a, W1b, b1b, gamma1, beta1, W2a, b2a, W2b, b2b, gamma2, beta2, Wout, bout):
    raise NotImplementedError("write your pallas kernel here")


# speedup vs baseline: 4.5685x; 4.5685x over previous
import functools

import jax
import jax.numpy as jnp
from jax import lax
from jax.experimental import pallas as pl
from jax.experimental.pallas import tpu as pltpu
from jax.experimental.pallas import tpu_sc as plsc

N = 10000
E = 320000
D = 128
NC = 2
NS = 16
NW = NC * NS
CH = 80

NPAD = 10240
GPT = NPAD // NW
GCH = GPT // CH

EPT = E // NW
ECH = EPT // CH
RPT = NPAD // NS

_mesh = plsc.VectorSubcoreMesh(core_axis_name="c", subcore_axis_name="s")


@functools.partial(
    pl.kernel,
    out_type=jax.ShapeDtypeStruct((NPAD, D), jnp.float32),
    mesh=_mesh,
    scratch_types=[
        pltpu.VMEM((CH,), jnp.int32),
        pltpu.VMEM((CH, D), jnp.float32),
        pltpu.SemaphoreType.DMA,
    ],
)
def _sc_gather_h0(emb_hbm, xt_hbm, out_hbm, tidx, rows, sem):
    c = lax.axis_index("c")
    s = lax.axis_index("s")
    base = (c * NS + s) * GPT

    def step(j, carry):
        off = pl.multiple_of(base + j * CH, 8)
        pltpu.sync_copy(xt_hbm.at[pl.ds(off, CH)], tidx)
        pltpu.async_copy(emb_hbm.at[tidx], rows, sem).wait()
        pltpu.sync_copy(rows, out_hbm.at[pl.ds(off, CH)])
        return carry

    lax.fori_loop(0, GCH, step, 0)


def _make_sc_agg(h_rows):
    @functools.partial(
        pl.kernel,
        out_type=jax.ShapeDtypeStruct((NC, NPAD, D), jnp.float32),
        mesh=_mesh,
        scratch_types=[
            pltpu.VMEM((CH,), jnp.int32),
            pltpu.VMEM((CH,), jnp.int32),
            pltpu.VMEM((CH, D), jnp.float32),
            pltpu.VMEM_SHARED((NPAD, D), jnp.float32),
            pltpu.SemaphoreType.DMA,
        ],
    )
    def _sc_agg(h_hbm, src_hbm, dst_hbm, zer_hbm, out_hbm, sidx, didx, rows,
                acc, sem):
        c = lax.axis_index("c")
        s = lax.axis_index("s")
        base = (c * NS + s) * EPT
        pltpu.sync_copy(zer_hbm, acc.at[pl.ds(s * RPT, RPT)])
        plsc.subcore_barrier()

        def step(j, carry):
            off = pl.multiple_of(base + j * CH, 8)
            pltpu.sync_copy(src_hbm.at[pl.ds(off, CH)], sidx)
            pltpu.sync_copy(dst_hbm.at[pl.ds(off, CH)], didx)
            pltpu.async_copy(h_hbm.at[sidx], rows, sem).wait()
            pltpu.sync_copy(rows, acc.at[didx], add=True)
            return carry

        lax.fori_loop(0, ECH, step, 0)
        plsc.subcore_barrier()
        pltpu.sync_copy(acc.at[pl.ds(s * RPT, RPT)],
                        out_hbm.at[c, pl.ds(s * RPT, RPT)])

    return _sc_agg


_sc_agg_l1 = _make_sc_agg(NPAD)
_sc_agg_l2 = _make_sc_agg(N)


def _tc_layer_body(h_ref, p_ref, wa_ref, ba_ref, wb_ref, bb_ref, g_ref,
                   be_ref, out_ref):
    x = h_ref[:N] + p_ref[0, :N] + p_ref[1, :N]
    z = jnp.maximum(
        jnp.dot(x, wa_ref[...], preferred_element_type=jnp.float32)
        + ba_ref[...], 0.0)
    t = (jnp.dot(z, wb_ref[...], preferred_element_type=jnp.float32)
         + bb_ref[...])
    mean = jnp.mean(t, axis=0, keepdims=True)
    var = jnp.mean((t - mean) ** 2, axis=0, keepdims=True)
    out_ref[...] = jnp.maximum(
        g_ref[...] * (t - mean) / jnp.sqrt(var + 1e-5) + be_ref[...], 0.0)


_tc_layer = pl.pallas_call(
    _tc_layer_body,
    out_shape=jax.ShapeDtypeStruct((N, D), jnp.float32),
)


def _tc_layer_out_body(h_ref, p_ref, wa_ref, ba_ref, wb_ref, bb_ref, g_ref,
                       be_ref, wo_ref, bo_ref, out_ref):
    x = h_ref[:N] + p_ref[0, :N] + p_ref[1, :N]
    z = jnp.maximum(
        jnp.dot(x, wa_ref[...], preferred_element_type=jnp.float32)
        + ba_ref[...], 0.0)
    t = (jnp.dot(z, wb_ref[...], preferred_element_type=jnp.float32)
         + bb_ref[...])
    mean = jnp.mean(t, axis=0, keepdims=True)
    var = jnp.mean((t - mean) ** 2, axis=0, keepdims=True)
    r = jnp.maximum(
        g_ref[...] * (t - mean) / jnp.sqrt(var + 1e-5) + be_ref[...], 0.0)
    out_ref[...] = (jnp.dot(r, wo_ref[...], preferred_element_type=jnp.float32)
                    + bo_ref[...])


def _tc_layer_out(a):
    return pl.pallas_call(
        _tc_layer_out_body,
        out_shape=jax.ShapeDtypeStruct((N, a), jnp.float32),
    )


def kernel(x_atom_type, edge_index, batch, emb, W1a, b1a, W1b, b1b, gamma1,
           beta1, W2a, b2a, W2b, b2b, gamma2, beta2, Wout, bout):
    src = edge_index[0]
    dst = edge_index[1]
    x_pad = jnp.concatenate(
        [x_atom_type.astype(jnp.int32),
         jnp.zeros((NPAD - N,), jnp.int32)])
    zer = jnp.zeros((RPT, D), jnp.float32)

    h0p = _sc_gather_h0(emb, x_pad)
    p1 = _sc_agg_l1(h0p, src, dst, zer)
    h1 = _tc_layer(h0p, p1, W1a, b1a[None], W1b, b1b[None],
                   gamma1[None], beta1[None])
    p2 = _sc_agg_l2(h1, src, dst, zer)
    logits = _tc_layer_out(Wout.shape[1])(
        h1, p2, W2a, b2a[None], W2b, b2b[None], gamma2[None],
        beta2[None], Wout, bout[None])
    return logits
